# Initial kernel scaffold; baseline (speedup 1.0000x reference)
import jax
import jax.numpy as jnp
from jax.experimental import pallas as pl


def kernel(x, edge_index, W_down, b_down, W_up, b_up):
    raise NotImplementedError("write your pallas kernel here")


# speedup vs baseline: 34.9673x; 34.9673x over previous
import functools

import jax
import jax.numpy as jnp
from jax import lax
from jax.experimental import pallas as pl
from jax.experimental.pallas import tpu as pltpu
from jax.experimental.pallas import tpu_sc as plsc

N = 10000
H = 128
BN = 32
NPAD = 10240
NC, NS = 2, 16
NW = NC * NS
G = 9
CHUNK = G * 128
NCHUNKS = 9
EPW = CHUNK * NCHUNKS
EPAD = NW * EPW
ROWS_PW = EPW // 128
DUMP = N
RPT = NPAD // NS
RB = 1024


def _sc_mesh():
    return plsc.VectorSubcoreMesh(
        core_axis_name="c", subcore_axis_name="s", num_cores=NC, num_subcores=NS
    )


_SC_PARAMS = pltpu.CompilerParams(use_tc_tiling_on_sc=False)


def _deg_pass(dst2d, ones_c, z16):
    @functools.partial(
        pl.kernel,
        out_type=jax.ShapeDtypeStruct((NC, NPAD, 16), jnp.float32),
        mesh=_sc_mesh(),
        scratch_types=[
            pltpu.VMEM((G, 1, 128), jnp.int32),
            pltpu.VMEM((CHUNK, 16), jnp.float32),
            pltpu.VMEM_SHARED((NPAD, 16), jnp.float32),
        ],
        compiler_params=_SC_PARAMS,
    )
    def deg_kernel(dst_hbm, ones_hbm, z_hbm, out_hbm, dstv, rows, acc):
        c = lax.axis_index("c")
        s = lax.axis_index("s")
        w = c * NS + s
        pltpu.sync_copy(z_hbm, acc.at[pl.ds(s * RPT, RPT)])
        pltpu.sync_copy(ones_hbm, rows)
        plsc.subcore_barrier()
        row0 = w * ROWS_PW

        def chunk(k, carry):
            r = row0 + k * G
            pltpu.sync_copy(dst_hbm.at[pl.ds(r, G)], dstv)
            for g in range(G):
                pltpu.sync_copy(
                    rows.at[pl.ds(g * 128, 128)], acc.at[dstv.at[g, 0]], add=True
                )
            return carry

        lax.fori_loop(0, NCHUNKS, chunk, 0)
        plsc.subcore_barrier()
        pltpu.sync_copy(
            acc.at[pl.ds(s * RPT, RPT)], out_hbm.at[c, pl.ds(s * RPT, RPT)]
        )

    return deg_kernel(dst2d, ones_c, z16)


def _conv_pass(table, src2d, dst2d, z32):
    @functools.partial(
        pl.kernel,
        out_type=jax.ShapeDtypeStruct((NC, NPAD, BN), jnp.float32),
        mesh=_sc_mesh(),
        scratch_types=[
            pltpu.VMEM((G, 1, 128), jnp.int32),
            pltpu.VMEM((G, 1, 128), jnp.int32),
            pltpu.VMEM((CHUNK, BN), jnp.float32),
            pltpu.VMEM_SHARED((NPAD, BN), jnp.float32),
            pltpu.SemaphoreType.DMA,
        ],
        compiler_params=_SC_PARAMS,
    )
    def conv_kernel(table_hbm, src_hbm, dst_hbm, z_hbm, out_hbm,
                    srcv, dstv, rows, acc, sem):
        c = lax.axis_index("c")
        s = lax.axis_index("s")
        w = c * NS + s
        pltpu.sync_copy(z_hbm, acc.at[pl.ds(s * RPT, RPT)])
        plsc.subcore_barrier()
        row0 = w * ROWS_PW

        def chunk(k, carry):
            r = row0 + k * G
            pltpu.sync_copy(src_hbm.at[pl.ds(r, G)], srcv)
            pltpu.sync_copy(dst_hbm.at[pl.ds(r, G)], dstv)
            descs = [
                pltpu.async_copy(
                    table_hbm.at[srcv.at[g, 0]], rows.at[pl.ds(g * 128, 128)], sem
                )
                for g in range(G)
            ]
            for d in descs:
                d.wait()
            for g in range(G):
                pltpu.sync_copy(
                    rows.at[pl.ds(g * 128, 128)], acc.at[dstv.at[g, 0]], add=True
                )
            return carry

        lax.fori_loop(0, NCHUNKS, chunk, 0)
        plsc.subcore_barrier()
        pltpu.sync_copy(
            acc.at[pl.ds(s * RPT, RPT)], out_hbm.at[c, pl.ds(s * RPT, RPT)]
        )

    return conv_kernel(table, src2d, dst2d, z32)


def _tc_down(degp, x_pad, w_down):
    def body(degp_ref, x_ref, wd_ref, h0s_ref, dis_ref):
        deg = degp_ref[0] + degp_ref[1]
        dis = jnp.where(deg > 0.0, lax.rsqrt(jnp.maximum(deg, 1.0)), 0.0)
        h0 = lax.dot_general(
            x_ref[...], wd_ref[...], (((1,), (1,)), ((), ())),
            preferred_element_type=jnp.float32,
        )
        h0s_ref[...] = h0 * dis[:, :1]
        dis_ref[...] = dis

    return pl.pallas_call(
        body,
        grid=(NPAD // RB,),
        in_specs=[
            pl.BlockSpec((NC, RB, 16), lambda i: (0, i, 0)),
            pl.BlockSpec((RB, H), lambda i: (i, 0)),
            pl.BlockSpec((BN, H), lambda i: (0, 0)),
        ],
        out_specs=[
            pl.BlockSpec((RB, BN), lambda i: (i, 0)),
            pl.BlockSpec((RB, 16), lambda i: (i, 0)),
        ],
        out_shape=[
            jax.ShapeDtypeStruct((NPAD, BN), jnp.float32),
            jax.ShapeDtypeStruct((NPAD, 16), jnp.float32),
        ],
    )(degp, x_pad, w_down)


def _tc_mid(m1p, dis16, b_down_row):
    def body(m1p_ref, dis_ref, b_ref, hs_ref):
        dis = dis_ref[...][:, :1]
        m1 = (m1p_ref[0] + m1p_ref[1]) * dis
        hs_ref[...] = jnp.maximum(m1 + b_ref[...], 0.0) * dis

    return pl.pallas_call(
        body,
        grid=(NPAD // RB,),
        in_specs=[
            pl.BlockSpec((NC, RB, BN), lambda i: (0, i, 0)),
            pl.BlockSpec((RB, 16), lambda i: (i, 0)),
            pl.BlockSpec((1, BN), lambda i: (0, 0)),
        ],
        out_specs=pl.BlockSpec((RB, BN), lambda i: (i, 0)),
        out_shape=jax.ShapeDtypeStruct((NPAD, BN), jnp.float32),
    )(m1p, dis16, b_down_row)


def _tc_up(m2p, dis16, w_up, b_up_row, x_pad):
    def body(m2p_ref, dis_ref, wu_ref, b_ref, x_ref, out_ref):
        m2 = (m2p_ref[0] + m2p_ref[1]) * dis_ref[...][:, :1]
        y = lax.dot_general(
            m2, wu_ref[...], (((1,), (1,)), ((), ())),
            preferred_element_type=jnp.float32,
        )
        out_ref[...] = y + b_ref[...] + x_ref[...]

    return pl.pallas_call(
        body,
        grid=(NPAD // RB,),
        in_specs=[
            pl.BlockSpec((NC, RB, BN), lambda i: (0, i, 0)),
            pl.BlockSpec((RB, 16), lambda i: (i, 0)),
            pl.BlockSpec((H, BN), lambda i: (0, 0)),
            pl.BlockSpec((1, H), lambda i: (0, 0)),
            pl.BlockSpec((RB, H), lambda i: (i, 0)),
        ],
        out_specs=pl.BlockSpec((RB, H), lambda i: (i, 0)),
        out_shape=jax.ShapeDtypeStruct((NPAD, H), jnp.float32),
    )(m2p, dis16, w_up, b_up_row, x_pad)


def kernel(x, edge_index, W_down, b_down, W_up, b_up):
    f32 = jnp.float32
    e = edge_index.shape[1]
    npadding = EPAD - e - N
    loop = jnp.arange(N, dtype=jnp.int32)
    fill = DUMP + (jnp.arange(npadding, dtype=jnp.int32) % (NPAD - N))
    src2d = jnp.concatenate([edge_index[0], loop, fill]).reshape(-1, 1, 128)
    dst2d = jnp.concatenate([edge_index[1], loop, fill]).reshape(-1, 1, 128)
    x_pad = jnp.pad(x, ((0, NPAD - N), (0, 0)))
    z16 = jnp.zeros((RPT, 16), f32)
    z32 = jnp.zeros((RPT, BN), f32)
    ones_c = jnp.ones((CHUNK, 16), f32)

    degp = _deg_pass(dst2d, ones_c, z16)
    h0s, dis16 = _tc_down(degp, x_pad, W_down)
    m1p = _conv_pass(h0s, src2d, dst2d, z32)
    hs = _tc_mid(m1p, dis16, b_down.reshape(1, BN))
    m2p = _conv_pass(hs, src2d, dst2d, z32)
    out = _tc_up(m2p, dis16, W_up, b_up.reshape(1, H), x_pad)
    return out[:N]


# speedup vs baseline: 42.4145x; 1.2130x over previous
import functools

import jax
import jax.numpy as jnp
from jax import lax
from jax.experimental import pallas as pl
from jax.experimental.pallas import tpu as pltpu
from jax.experimental.pallas import tpu_sc as plsc

N = 10000
H = 128
BN = 32
NPAD = 10240
NC, NS = 2, 16
NW = NC * NS
G = 9
CHUNK = G * 128
NCHUNKS = 9
EPW = CHUNK * NCHUNKS
EPAD = NW * EPW
ROWS_PW = EPW // 128
DUMP = N
RPT = NPAD // NS
RB = 1024


def _sc_mesh():
    return plsc.VectorSubcoreMesh(
        core_axis_name="c", subcore_axis_name="s", num_cores=NC, num_subcores=NS
    )


_SC_PARAMS = pltpu.CompilerParams(use_tc_tiling_on_sc=False)


def _deg_pass(dst2d, ones_c, z16):
    @functools.partial(
        pl.kernel,
        out_type=jax.ShapeDtypeStruct((NC, NPAD, 16), jnp.float32),
        mesh=_sc_mesh(),
        scratch_types=[
            pltpu.VMEM((ROWS_PW, 1, 128), jnp.int32),
            pltpu.VMEM((128, 16), jnp.float32),
            pltpu.VMEM_SHARED((NPAD, 16), jnp.float32),
            pltpu.SemaphoreType.DMA,
        ],
        compiler_params=_SC_PARAMS,
    )
    def deg_kernel(dst_hbm, ones_hbm, z_hbm, out_hbm, dstv, ones_v, acc, sem):
        c = lax.axis_index("c")
        s = lax.axis_index("s")
        w = c * NS + s
        pltpu.sync_copy(z_hbm, acc.at[pl.ds(s * RPT, RPT)])
        pltpu.sync_copy(ones_hbm, ones_v)
        pltpu.sync_copy(dst_hbm.at[pl.ds(w * ROWS_PW, ROWS_PW)], dstv)
        plsc.subcore_barrier()
        descs = []
        for r in range(ROWS_PW):
            if r >= 12:
                descs[r - 12].wait()
            descs.append(
                pltpu.async_copy(ones_v, acc.at[dstv.at[r, 0]], sem, add=True)
            )
        for d in descs[-12:]:
            d.wait()
        plsc.subcore_barrier()
        pltpu.sync_copy(
            acc.at[pl.ds(s * RPT, RPT)], out_hbm.at[c, pl.ds(s * RPT, RPT)]
        )

    return deg_kernel(dst2d, ones_c, z16)


def _conv_pass(table, src2d, dst2d, z32):
    @functools.partial(
        pl.kernel,
        out_type=jax.ShapeDtypeStruct((NC, NPAD, BN), jnp.float32),
        mesh=_sc_mesh(),
        scratch_types=[
            pltpu.VMEM((ROWS_PW, 1, 128), jnp.int32),
            pltpu.VMEM((ROWS_PW, 1, 128), jnp.int32),
            pltpu.VMEM((CHUNK, BN), jnp.float32),
            pltpu.VMEM((CHUNK, BN), jnp.float32),
            pltpu.SemaphoreType.DMA,
            pltpu.SemaphoreType.DMA,
            pltpu.SemaphoreType.DMA,
            pltpu.SemaphoreType.DMA,
            pltpu.VMEM_SHARED((NPAD, BN), jnp.float32),
        ],
        compiler_params=_SC_PARAMS,
    )
    def conv_kernel(table_hbm, src_hbm, dst_hbm, z_hbm, out_hbm,
                    srcv, dstv, rows0, rows1, gsem0, gsem1, ssem0, ssem1, acc):
        c = lax.axis_index("c")
        s = lax.axis_index("s")
        w = c * NS + s
        pltpu.sync_copy(z_hbm, acc.at[pl.ds(s * RPT, RPT)])
        pltpu.sync_copy(src_hbm.at[pl.ds(w * ROWS_PW, ROWS_PW)], srcv)
        pltpu.sync_copy(dst_hbm.at[pl.ds(w * ROWS_PW, ROWS_PW)], dstv)
        plsc.subcore_barrier()
        rows = (rows0, rows1)
        gsem = (gsem0, gsem1)
        ssem = (ssem0, ssem1)

        def fire_gather(k):
            b = k & 1
            return [
                pltpu.async_copy(
                    table_hbm.at[srcv.at[k * G + g, 0]],
                    rows[b].at[pl.ds(g * 128, 128)],
                    gsem[b],
                )
                for g in range(G)
            ]

        def fire_scatter(k):
            b = k & 1
            return [
                pltpu.async_copy(
                    rows[b].at[pl.ds(g * 128, 128)],
                    acc.at[dstv.at[k * G + g, 0]],
                    ssem[b],
                    add=True,
                )
                for g in range(G)
            ]

        gd = {0: fire_gather(0)}
        sd = {}
        for k in range(NCHUNKS):
            for d in gd[k]:
                d.wait()
            sd[k] = fire_scatter(k)
            if k + 1 < NCHUNKS:
                if k - 1 >= 0:
                    for d in sd[k - 1]:
                        d.wait()
                gd[k + 1] = fire_gather(k + 1)
        for k in (NCHUNKS - 2, NCHUNKS - 1):
            for d in sd[k]:
                d.wait()
        plsc.subcore_barrier()
        pltpu.sync_copy(
            acc.at[pl.ds(s * RPT, RPT)], out_hbm.at[c, pl.ds(s * RPT, RPT)]
        )

    return conv_kernel(table, src2d, dst2d, z32)


def _tc_down(degp, x_pad, w_down):
    def body(degp_ref, x_ref, wd_ref, h0s_ref, dis_ref):
        deg = degp_ref[0] + degp_ref[1]
        dis = jnp.where(deg > 0.0, lax.rsqrt(jnp.maximum(deg, 1.0)), 0.0)
        h0 = lax.dot_general(
            x_ref[...], wd_ref[...], (((1,), (1,)), ((), ())),
            preferred_element_type=jnp.float32,
        )
        h0s_ref[...] = h0 * dis[:, :1]
        dis_ref[...] = dis

    return pl.pallas_call(
        body,
        grid=(NPAD // RB,),
        in_specs=[
            pl.BlockSpec((NC, RB, 16), lambda i: (0, i, 0)),
            pl.BlockSpec((RB, H), lambda i: (i, 0)),
            pl.BlockSpec((BN, H), lambda i: (0, 0)),
        ],
        out_specs=[
            pl.BlockSpec((RB, BN), lambda i: (i, 0)),
            pl.BlockSpec((RB, 16), lambda i: (i, 0)),
        ],
        out_shape=[
            jax.ShapeDtypeStruct((NPAD, BN), jnp.float32),
            jax.ShapeDtypeStruct((NPAD, 16), jnp.float32),
        ],
    )(degp, x_pad, w_down)


def _tc_mid(m1p, dis16, b_down_row):
    def body(m1p_ref, dis_ref, b_ref, hs_ref):
        dis = dis_ref[...][:, :1]
        m1 = (m1p_ref[0] + m1p_ref[1]) * dis
        hs_ref[...] = jnp.maximum(m1 + b_ref[...], 0.0) * dis

    return pl.pallas_call(
        body,
        grid=(NPAD // RB,),
        in_specs=[
            pl.BlockSpec((NC, RB, BN), lambda i: (0, i, 0)),
            pl.BlockSpec((RB, 16), lambda i: (i, 0)),
            pl.BlockSpec((1, BN), lambda i: (0, 0)),
        ],
        out_specs=pl.BlockSpec((RB, BN), lambda i: (i, 0)),
        out_shape=jax.ShapeDtypeStruct((NPAD, BN), jnp.float32),
    )(m1p, dis16, b_down_row)


def _tc_up(m2p, dis16, w_up, b_up_row, x_pad):
    def body(m2p_ref, dis_ref, wu_ref, b_ref, x_ref, out_ref):
        m2 = (m2p_ref[0] + m2p_ref[1]) * dis_ref[...][:, :1]
        y = lax.dot_general(
            m2, wu_ref[...], (((1,), (1,)), ((), ())),
            preferred_element_type=jnp.float32,
        )
        out_ref[...] = y + b_ref[...] + x_ref[...]

    return pl.pallas_call(
        body,
        grid=(NPAD // RB,),
        in_specs=[
            pl.BlockSpec((NC, RB, BN), lambda i: (0, i, 0)),
            pl.BlockSpec((RB, 16), lambda i: (i, 0)),
            pl.BlockSpec((H, BN), lambda i: (0, 0)),
            pl.BlockSpec((1, H), lambda i: (0, 0)),
            pl.BlockSpec((RB, H), lambda i: (i, 0)),
        ],
        out_specs=pl.BlockSpec((RB, H), lambda i: (i, 0)),
        out_shape=jax.ShapeDtypeStruct((NPAD, H), jnp.float32),
    )(m2p, dis16, w_up, b_up_row, x_pad)


def kernel(x, edge_index, W_down, b_down, W_up, b_up):
    f32 = jnp.float32
    e = edge_index.shape[1]
    npadding = EPAD - e - N
    loop = jnp.arange(N, dtype=jnp.int32)
    fill = DUMP + (jnp.arange(npadding, dtype=jnp.int32) % (NPAD - N))
    src2d = jnp.concatenate([edge_index[0], loop, fill]).reshape(-1, 1, 128)
    dst2d = jnp.concatenate([edge_index[1], loop, fill]).reshape(-1, 1, 128)
    x_pad = jnp.pad(x, ((0, NPAD - N), (0, 0)))
    z16 = jnp.zeros((RPT, 16), f32)
    z32 = jnp.zeros((RPT, BN), f32)
    ones_c = jnp.ones((128, 16), f32)

    degp = _deg_pass(dst2d, ones_c, z16)
    h0s, dis16 = _tc_down(degp, x_pad, W_down)
    m1p = _conv_pass(h0s, src2d, dst2d, z32)
    hs = _tc_mid(m1p, dis16, b_down.reshape(1, BN))
    m2p = _conv_pass(hs, src2d, dst2d, z32)
    out = _tc_up(m2p, dis16, W_up, b_up.reshape(1, H), x_pad)
    return out[:N]
